# baseline (speedup 1.0000x reference)
import dataclasses

import jax
import jax.numpy as jnp
from jax import lax
from jax.experimental import pallas as pl
from jax.experimental.pallas import tpu as pltpu
from jax.experimental.pallas import tpu_sc as plsc

T = 64
D_OUT = 128
NCORES = 2
NSUB = 16
NW = NCORES * NSUB
KT = 2


def _stochastic_round_delays(log_delay, N, C):
    D = log_delay.shape[0]
    delay = jnp.concatenate([jnp.exp(log_delay), jnp.exp(log_delay[::-1])],
                            axis=1)
    db = jnp.broadcast_to(delay[None, None, :, :], (N, C, D, 2))
    fl = jnp.floor(db)
    p = db - fl
    bern = jax.random.bernoulli(jax.random.key(42), p)
    return jnp.where(bern, fl + 1.0, fl).astype(jnp.int32)


def _sc_body(x_hbm, rd_hbm, lw_hbm, mr_hbm, out_hbm,
             xstage, xbuf, rdbuf, wbuf, mrbuf, sbuf, capsm,
             tile0, tile1, sem0, sem1):
    ch_per_w = xstage.shape[0]
    wid = lax.axis_index("c") * NSUB + lax.axis_index("s")
    base_ch = wid * ch_per_w

    pltpu.sync_copy(x_hbm.at[pl.ds(base_ch, ch_per_w)], xstage)
    pltpu.sync_copy(rd_hbm.at[pl.ds(base_ch, ch_per_w)], rdbuf)
    pltpu.sync_copy(lw_hbm, wbuf)
    pltpu.sync_copy(mr_hbm, mrbuf)
    wv = jnp.exp(wbuf[...])
    mr = jnp.max(mrbuf[...])

    iota16 = lax.broadcasted_iota(jnp.int32, (16,), 0)
    nck = T // 16

    @pl.loop(0, ch_per_w)
    def _(i):
        for j in range(2):
            for k in range(nck):
                v = xstage[i, j, pl.ds(16 * k, 16)] * wv
                xbuf[i, j, pl.ds(16 * k, 16)] = v
                xbuf[i, j, pl.ds(T + 16 * k, 16)] = v
        for j in range(2):
            m = jnp.max(xbuf[i, j, pl.ds(0, 16)])
            for k in range(1, nck):
                m = jnp.maximum(m, jnp.max(xbuf[i, j, pl.ds(16 * k, 16)]))
            best = jnp.int32(T)
            for k in range(nck):
                ck = xbuf[i, j, pl.ds(16 * k, 16)]
                idxs = jnp.where(ck == m, iota16 + 16 * k, jnp.int32(127))
                best = jnp.minimum(best, jnp.min(idxs))
            capsm[2 * i + j] = jnp.int32(T - 1) - best
        for k in range(nck):
            a0 = xbuf[i, 0, pl.ds(16 * k, 16)]
            a1 = xbuf[i, 1, pl.ds(16 * k, 16)]
            p0 = xbuf[i, 0, pl.ds(16 * k + T - 1, 16)]
            p1 = xbuf[i, 1, pl.ds(16 * k + T - 1, 16)]
            sbuf[i, 0, pl.ds(16 * k, 16)] = a0 + a1
            sbuf[i, 1, pl.ds(16 * k, 16)] = p0 - a0
            sbuf[i, 2, pl.ds(16 * k, 16)] = p1 - a1

    tiles = (tile0, tile1)
    sems = (sem0, sem1)

    def fast_tile(tile, t0):
        @plsc.parallel_loop(0, ch_per_w, step=1)
        def _(i):
            cap0 = capsm[2 * i]
            cap1 = capsm[2 * i + 1]
            i16 = jnp.broadcast_to(i, (16,)).astype(jnp.int32)
            z16 = jnp.zeros((16,), jnp.int32)
            o16 = jnp.ones((16,), jnp.int32)
            two16 = jnp.full((16,), 2, jnp.int32)
            ms = []
            for k8 in range(D_OUT // 16):
                dsl = pl.ds(16 * k8, 16)
                r0 = jnp.minimum(rdbuf[i, 0, dsl], cap0)
                r1 = jnp.minimum(rdbuf[i, 1, dsl], cap1)
                ms.append(((r0 == 1).astype(jnp.float32),
                           (r1 == 1).astype(jnp.float32)))
            for tt in range(KT):
                t16 = jnp.broadcast_to(t0 + tt, (16,)).astype(jnp.int32)
                sv = plsc.load_gather(sbuf, [i16, z16, t16])
                d0v = plsc.load_gather(sbuf, [i16, o16, t16])
                d1v = plsc.load_gather(sbuf, [i16, two16, t16])
                for k8 in range(D_OUT // 16):
                    m0, m1 = ms[k8]
                    tile[tt, i, pl.ds(16 * k8, 16)] = sv + m0 * d0v + m1 * d1v

    def slow_tile(tile, t0):
        @plsc.parallel_loop(0, ch_per_w, step=1)
        def _(i):
            cap0 = capsm[2 * i]
            cap1 = capsm[2 * i + 1]
            i16 = jnp.broadcast_to(i, (16,)).astype(jnp.int32)
            z16 = jnp.zeros((16,), jnp.int32)
            o16 = jnp.ones((16,), jnp.int32)
            for k8 in range(D_OUT // 16):
                dsl = pl.ds(16 * k8, 16)
                b0 = (T + t0) - jnp.minimum(rdbuf[i, 0, dsl], cap0)
                b1 = (T + t0) - jnp.minimum(rdbuf[i, 1, dsl], cap1)
                for tt in range(KT):
                    g0 = plsc.load_gather(xbuf, [i16, z16, b0 + tt])
                    g1 = plsc.load_gather(xbuf, [i16, o16, b1 + tt])
                    tile[tt, i, dsl] = g0 + g1

    @pl.loop(0, T // KT // 2)
    def _(gp):
        for b in range(2):
            g = gp * 2 + b
            t0 = g * KT
            dst = out_hbm.at[g, :, pl.ds(base_ch, ch_per_w), :]

            @pl.when(gp > 0)
            def _():
                pltpu.make_async_copy(tiles[b], dst, sems[b]).wait()

            @pl.when(mr <= 1)
            def _():
                fast_tile(tiles[b], t0)

            @pl.when(mr > 1)
            def _():
                slow_tile(tiles[b], t0)

            pltpu.async_copy(tiles[b], dst, sems[b])

    for b in range(2):
        pltpu.make_async_copy(
            tiles[b],
            out_hbm.at[0, :, pl.ds(base_ch, ch_per_w), :],
            sems[b]).wait()


def kernel(input, log_delay, log_weight):
    Tt, N, C, _ = input.shape
    D = log_delay.shape[0]
    NC = N * C
    ch_per_w = NC // NW

    rd_pre = _stochastic_round_delays(log_delay, N, C)
    rdf = jnp.transpose(rd_pre, (0, 1, 3, 2)).reshape(NC, 2, D)
    xf = jnp.transpose(input, (1, 2, 3, 0)).reshape(NC, 2, Tt)
    lwv = jnp.full((16,), log_weight, jnp.float32)
    mrv = jnp.full((16,), jnp.max(rd_pre), jnp.int32)

    mesh = plsc.VectorSubcoreMesh(core_axis_name="c", subcore_axis_name="s")
    cp = pltpu.CompilerParams()
    if "needs_layout_passes" in pltpu.CompilerParams.__dataclass_fields__:
        cp = dataclasses.replace(cp, needs_layout_passes=False)
    run = pl.kernel(
        _sc_body,
        out_type=jax.ShapeDtypeStruct((Tt // KT, KT, NC, D), jnp.float32),
        mesh=mesh,
        scratch_types=[
            pltpu.VMEM((ch_per_w, 2, Tt), jnp.float32),
            pltpu.VMEM((ch_per_w, 2, 2 * Tt), jnp.float32),
            pltpu.VMEM((ch_per_w, 2, D), jnp.int32),
            pltpu.VMEM((16,), jnp.float32),
            pltpu.VMEM((16,), jnp.int32),
            pltpu.VMEM((ch_per_w, 4, 2 * Tt), jnp.float32),
            pltpu.SMEM((2 * ch_per_w,), jnp.int32),
            pltpu.VMEM((KT, ch_per_w, D), jnp.float32),
            pltpu.VMEM((KT, ch_per_w, D), jnp.float32),
            pltpu.SemaphoreType.DMA,
            pltpu.SemaphoreType.DMA,
        ],
        compiler_params=cp,
    )
    out = run(xf, rdf, lwv, mrv)
    return out.reshape(Tt, N, C, D)


# speedup vs baseline: 1.0536x; 1.0536x over previous
import dataclasses

import jax
import jax.numpy as jnp
from jax import lax
from jax.experimental import pallas as pl
from jax.experimental.pallas import tpu as pltpu
from jax.experimental.pallas import tpu_sc as plsc

T = 64
D_OUT = 128
NCORES = 2
NSUB = 16
NW = NCORES * NSUB
KT = 4


def _stochastic_round_delays(log_delay, N, C):
    D = log_delay.shape[0]
    delay = jnp.concatenate([jnp.exp(log_delay), jnp.exp(log_delay[::-1])],
                            axis=1)
    db = jnp.broadcast_to(delay[None, None, :, :], (N, C, D, 2))
    fl = jnp.floor(db)
    p = db - fl
    bern = jax.random.bernoulli(jax.random.key(42), p)
    return jnp.where(bern, fl + 1.0, fl).astype(jnp.int32)


def _sc_body(x_hbm, rd_hbm, lw_hbm, mr_hbm, out_hbm,
             xbuf, rdbuf, wbuf, mrbuf, sbuf, capsm,
             tile0, tile1, sem0, sem1):
    ch_per_w = xbuf.shape[0]
    wid = lax.axis_index("c") * NSUB + lax.axis_index("s")
    base_ch = wid * ch_per_w

    pltpu.sync_copy(x_hbm.at[pl.ds(base_ch, ch_per_w)], xbuf)
    pltpu.sync_copy(rd_hbm.at[pl.ds(base_ch, ch_per_w)], rdbuf)
    pltpu.sync_copy(lw_hbm, wbuf)
    pltpu.sync_copy(mr_hbm, mrbuf)
    wv = jnp.exp(wbuf[...])
    mr = jnp.max(mrbuf[...])

    iota16 = lax.broadcasted_iota(jnp.int32, (16,), 0)
    nck = T // 16

    @pl.loop(0, ch_per_w)
    def _(i):
        for j in range(2):
            for k in range(2 * nck):
                sl = pl.ds(16 * k, 16)
                xbuf[i, j, sl] = xbuf[i, j, sl] * wv
        for j in range(2):
            m = jnp.max(xbuf[i, j, pl.ds(0, 16)])
            for k in range(1, nck):
                m = jnp.maximum(m, jnp.max(xbuf[i, j, pl.ds(16 * k, 16)]))
            best = jnp.int32(T)
            for k in range(nck):
                ck = xbuf[i, j, pl.ds(16 * k, 16)]
                idxs = jnp.where(ck == m, iota16 + 16 * k, jnp.int32(127))
                best = jnp.minimum(best, jnp.min(idxs))
            capsm[2 * i + j] = jnp.int32(T - 1) - best
        for k in range(nck):
            a0 = xbuf[i, 0, pl.ds(16 * k, 16)]
            a1 = xbuf[i, 1, pl.ds(16 * k, 16)]
            p0 = xbuf[i, 0, pl.ds(16 * k + T - 1, 16)]
            p1 = xbuf[i, 1, pl.ds(16 * k + T - 1, 16)]
            sbuf[i, 0, pl.ds(16 * k, 16)] = a0 + a1
            sbuf[i, 0, pl.ds(T + 16 * k, 16)] = p0 - a0
            sbuf[i, 1, pl.ds(16 * k, 16)] = p1 - a1

    tiles = (tile0, tile1)
    sems = (sem0, sem1)

    def fast_tile(tile, t0):
        @plsc.parallel_loop(0, ch_per_w, step=1)
        def _(i):
            cap0 = capsm[2 * i]
            cap1 = capsm[2 * i + 1]
            i16 = jnp.broadcast_to(i, (16,)).astype(jnp.int32)
            z16 = jnp.zeros((16,), jnp.int32)
            o16 = jnp.ones((16,), jnp.int32)
            two16 = jnp.full((16,), 2, jnp.int32)
            ms = []
            for k8 in range(D_OUT // 16):
                dsl = pl.ds(16 * k8, 16)
                r0 = jnp.minimum(rdbuf[i, 0, dsl], cap0)
                r1 = jnp.minimum(rdbuf[i, 1, dsl], cap1)
                ms.append(((r0 == 1).astype(jnp.float32),
                           (r1 == 1).astype(jnp.float32)))
            for tt in range(KT):
                t16 = jnp.broadcast_to(t0 + tt, (16,)).astype(jnp.int32)
                sv = plsc.load_gather(sbuf, [i16, z16, t16])
                d0v = plsc.load_gather(sbuf, [i16, z16, t16 + T])
                d1v = plsc.load_gather(sbuf, [i16, o16, t16])
                for k8 in range(D_OUT // 16):
                    m0, m1 = ms[k8]
                    tile[tt, i, pl.ds(16 * k8, 16)] = sv + m0 * d0v + m1 * d1v

    def slow_tile(tile, t0):
        @plsc.parallel_loop(0, ch_per_w, step=1)
        def _(i):
            cap0 = capsm[2 * i]
            cap1 = capsm[2 * i + 1]
            i16 = jnp.broadcast_to(i, (16,)).astype(jnp.int32)
            z16 = jnp.zeros((16,), jnp.int32)
            o16 = jnp.ones((16,), jnp.int32)
            for k8 in range(D_OUT // 16):
                dsl = pl.ds(16 * k8, 16)
                b0 = (T + t0) - jnp.minimum(rdbuf[i, 0, dsl], cap0)
                b1 = (T + t0) - jnp.minimum(rdbuf[i, 1, dsl], cap1)
                for tt in range(KT):
                    g0 = plsc.load_gather(xbuf, [i16, z16, b0 + tt])
                    g1 = plsc.load_gather(xbuf, [i16, o16, b1 + tt])
                    tile[tt, i, dsl] = g0 + g1

    @pl.loop(0, T // KT // 2)
    def _(gp):
        for b in range(2):
            g = gp * 2 + b
            t0 = g * KT
            dst = out_hbm.at[g, :, pl.ds(base_ch, ch_per_w), :]

            @pl.when(gp > 0)
            def _():
                pltpu.make_async_copy(tiles[b], dst, sems[b]).wait()

            @pl.when(mr <= 1)
            def _():
                fast_tile(tiles[b], t0)

            @pl.when(mr > 1)
            def _():
                slow_tile(tiles[b], t0)

            pltpu.async_copy(tiles[b], dst, sems[b])

    for b in range(2):
        pltpu.make_async_copy(
            tiles[b],
            out_hbm.at[0, :, pl.ds(base_ch, ch_per_w), :],
            sems[b]).wait()


def kernel(input, log_delay, log_weight):
    Tt, N, C, _ = input.shape
    D = log_delay.shape[0]
    NC = N * C
    ch_per_w = NC // NW

    rd_pre = _stochastic_round_delays(log_delay, N, C)
    rdf = jnp.transpose(rd_pre, (0, 1, 3, 2)).reshape(NC, 2, D)
    xf = jnp.transpose(input, (1, 2, 3, 0)).reshape(NC, 2, Tt)
    xf = jnp.concatenate([xf, xf], axis=-1)
    lwv = jnp.full((16,), log_weight, jnp.float32)
    mrv = jnp.full((16,), jnp.max(rd_pre), jnp.int32)

    mesh = plsc.VectorSubcoreMesh(core_axis_name="c", subcore_axis_name="s")
    cp = pltpu.CompilerParams()
    if "needs_layout_passes" in pltpu.CompilerParams.__dataclass_fields__:
        cp = dataclasses.replace(cp, needs_layout_passes=False)
    run = pl.kernel(
        _sc_body,
        out_type=jax.ShapeDtypeStruct((Tt // KT, KT, NC, D), jnp.float32),
        mesh=mesh,
        scratch_types=[
            pltpu.VMEM((ch_per_w, 2, 2 * Tt), jnp.float32),
            pltpu.VMEM((ch_per_w, 2, D), jnp.int32),
            pltpu.VMEM((16,), jnp.float32),
            pltpu.VMEM((16,), jnp.int32),
            pltpu.VMEM((ch_per_w, 2, 2 * Tt), jnp.float32),
            pltpu.SMEM((2 * ch_per_w,), jnp.int32),
            pltpu.VMEM((KT, ch_per_w, D), jnp.float32),
            pltpu.VMEM((KT, ch_per_w, D), jnp.float32),
            pltpu.SemaphoreType.DMA,
            pltpu.SemaphoreType.DMA,
        ],
        compiler_params=cp,
    )
    out = run(xf, rdf, lwv, mrv)
    return out.reshape(Tt, N, C, D)


# speedup vs baseline: 1.1155x; 1.0587x over previous
import dataclasses

import jax
import jax.numpy as jnp
from jax import lax
from jax.experimental import pallas as pl
from jax.experimental.pallas import tpu as pltpu
from jax.experimental.pallas import tpu_sc as plsc

T = 64
D_OUT = 128
NCORES = 2
NSUB = 16
NW = NCORES * NSUB
KCH = 4


def _stochastic_round_delays(log_delay, N, C):
    D = log_delay.shape[0]
    delay = jnp.concatenate([jnp.exp(log_delay), jnp.exp(log_delay[::-1])],
                            axis=1)
    db = jnp.broadcast_to(delay[None, None, :, :], (N, C, D, 2))
    fl = jnp.floor(db)
    p = db - fl
    bern = jax.random.bernoulli(jax.random.key(42), p)
    return jnp.where(bern, fl + 1.0, fl).astype(jnp.int32)


def _sc_body(x_hbm, rd_hbm, lw_hbm, out_hbm,
             xstage, xbuf, rdbuf, wbuf, tile0, tile1, sem0, sem1):
    ch_per_w = xbuf.shape[0]
    wid = lax.axis_index("c") * NSUB + lax.axis_index("s")
    base_ch = wid * ch_per_w

    pltpu.sync_copy(x_hbm.at[pl.ds(base_ch, ch_per_w)], xstage)
    pltpu.sync_copy(rd_hbm.at[pl.ds(base_ch, ch_per_w)], rdbuf)
    pltpu.sync_copy(lw_hbm, wbuf)
    wv = jnp.exp(wbuf[...])

    @pl.loop(0, ch_per_w)
    def _(i):
        for j in range(2):
            for k in range(T // 16):
                v = xstage[i, j, pl.ds(16 * k, 16)] * wv
                xbuf[i, j, pl.ds(16 * k, 16)] = v
                xbuf[i, j, pl.ds(T + 16 * k, 16)] = v

    iota16 = lax.broadcasted_iota(jnp.int32, (16,), 0)
    tiles = (tile0, tile1)
    sems = (sem0, sem1)
    ngroups = ch_per_w // KCH

    def compute_channel(tile, i, cc):
        caps = []
        for j in range(2):
            m = jnp.max(xbuf[i, j, pl.ds(0, 16)])
            for k in range(1, T // 16):
                m = jnp.maximum(m, jnp.max(xbuf[i, j, pl.ds(16 * k, 16)]))
            best = jnp.int32(T)
            for k in range(T // 16):
                ck = xbuf[i, j, pl.ds(16 * k, 16)]
                idxs = jnp.where(ck == m, iota16 + 16 * k, jnp.int32(127))
                best = jnp.minimum(best, jnp.min(idxs))
            caps.append(jnp.int32(T - 1) - best)
        i16 = jnp.broadcast_to(i, (16,)).astype(jnp.int32)
        z16 = jnp.zeros((16,), jnp.int32)
        o16 = jnp.ones((16,), jnp.int32)
        bs = []
        for k8 in range(D_OUT // 16):
            dsl = pl.ds(16 * k8, 16)
            rd0 = jnp.minimum(rdbuf[i, 0, dsl], caps[0]) & (T - 1)
            rd1 = jnp.minimum(rdbuf[i, 1, dsl], caps[1]) & (T - 1)
            bs.append((T - rd0, T - rd1))

        @plsc.parallel_loop(0, T, step=1, unroll=4)
        def _(t):
            for k8 in range(D_OUT // 16):
                b0, b1 = bs[k8]
                g0 = plsc.load_gather(xbuf, [i16, z16, b0 + t])
                g1 = plsc.load_gather(xbuf, [i16, o16, b1 + t])
                tile[t, cc, pl.ds(16 * k8, 16)] = g0 + g1

    @pl.loop(0, ngroups // 2)
    def _(gp):
        for b in range(2):
            g = gp * 2 + b
            ch0 = base_ch + g * KCH

            @pl.when(gp > 0)
            def _():
                pltpu.make_async_copy(
                    tiles[b], out_hbm.at[:, pl.ds(ch0, KCH), :], sems[b]).wait()

            @pl.loop(0, KCH)
            def _(cc):
                compute_channel(tiles[b], g * KCH + cc, cc)

            pltpu.async_copy(tiles[b], out_hbm.at[:, pl.ds(ch0, KCH), :],
                             sems[b])

    for b in range(2):
        pltpu.make_async_copy(
            tiles[b], out_hbm.at[:, pl.ds(base_ch, KCH), :], sems[b]).wait()


def kernel(input, log_delay, log_weight):
    Tt, N, C, _ = input.shape
    D = log_delay.shape[0]
    NC = N * C
    ch_per_w = NC // NW

    rd_pre = _stochastic_round_delays(log_delay, N, C)
    rdf = jnp.transpose(rd_pre, (0, 1, 3, 2)).reshape(NC, 2, D)
    xf = jnp.transpose(input, (1, 2, 3, 0)).reshape(NC, 2, Tt)
    lwv = jnp.full((16,), log_weight, jnp.float32)

    mesh = plsc.VectorSubcoreMesh(core_axis_name="c", subcore_axis_name="s")
    cp = pltpu.CompilerParams()
    if "needs_layout_passes" in pltpu.CompilerParams.__dataclass_fields__:
        cp = dataclasses.replace(cp, needs_layout_passes=False)
    run = pl.kernel(
        _sc_body,
        out_type=jax.ShapeDtypeStruct((Tt, NC, D), jnp.float32),
        mesh=mesh,
        scratch_types=[
            pltpu.VMEM((ch_per_w, 2, Tt), jnp.float32),
            pltpu.VMEM((ch_per_w, 2, 2 * Tt), jnp.float32),
            pltpu.VMEM((ch_per_w, 2, D), jnp.int32),
            pltpu.VMEM((16,), jnp.float32),
            pltpu.VMEM((Tt, KCH, D), jnp.float32),
            pltpu.VMEM((Tt, KCH, D), jnp.float32),
            pltpu.SemaphoreType.DMA,
            pltpu.SemaphoreType.DMA,
        ],
        compiler_params=cp,
    )
    out = run(xf, rdf, lwv)
    return out.reshape(Tt, N, C, D)
